# Initial kernel scaffold; baseline (speedup 1.0000x reference)
import jax
import jax.numpy as jnp
from jax.experimental import pallas as pl


def kernel(x, mask, w_gate, W1, b1, W2, b2):
    raise NotImplementedError("write your pallas kernel here")


# speedup vs baseline: 1.2441x; 1.2441x over previous
import functools

import jax
import jax.numpy as jnp
from jax.experimental import pallas as pl
from jax.experimental.pallas import tpu as pltpu

LOSS_COEF = 0.01


def _moe_body(x_ref, m_ref, wg_ref, W1_ref, b1_ref, W2_ref, b2_ref,
              y_ref, loss_ref, gates_ref, imp_ref, *, E, T):
    t = pl.program_id(0)
    e = pl.program_id(1)
    Sb = x_ref.shape[0]

    @pl.when(e == 0)
    def _gating():
        x = x_ref[...]
        logits = jnp.dot(x, wg_ref[...], preferred_element_type=jnp.float32)
        idx = jax.lax.broadcasted_iota(jnp.int32, (Sb, E), 1)
        m1 = jnp.max(logits, axis=1, keepdims=True)
        i1 = jnp.min(jnp.where(logits == m1, idx, E), axis=1, keepdims=True)
        masked = jnp.where(idx == i1, -jnp.inf, logits)
        m2 = jnp.max(masked, axis=1, keepdims=True)
        i2 = jnp.min(jnp.where(masked == m2, idx, E), axis=1, keepdims=True)
        b = jnp.exp(m2 - m1)
        denom = 1.0 + b
        gates = (1.0 / denom) * (idx == i1) + (b / denom) * (idx == i2)
        gates = gates * m_ref[...]
        gates_ref[...] = gates
        imp_part = jnp.sum(gates, axis=0, keepdims=True)

        @pl.when(t == 0)
        def _():
            imp_ref[...] = imp_part

        @pl.when(t > 0)
        def _():
            imp_ref[...] += imp_part

    x = x_ref[...]
    h = jnp.dot(x, W1_ref[0], preferred_element_type=jnp.float32) + b1_ref[0]
    h = jnp.maximum(h, 0.0)
    o = jnp.dot(h, W2_ref[0], preferred_element_type=jnp.float32) + b2_ref[0]
    idx = jax.lax.broadcasted_iota(jnp.int32, (Sb, E), 1)
    g = jnp.sum(gates_ref[...] * (idx == e), axis=1, keepdims=True)
    contrib = g * o

    @pl.when(e == 0)
    def _init():
        y_ref[...] = contrib

    @pl.when(e > 0)
    def _acc():
        y_ref[...] += contrib

    @pl.when(e == E - 1)
    def _fin():
        y_ref[...] = jax.nn.sigmoid(y_ref[...]) + x

    @pl.when((t == T - 1) & (e == E - 1))
    def _loss():
        imp = imp_ref[...]
        mean = jnp.mean(imp, axis=1, keepdims=True)
        var = jnp.sum((imp - mean) ** 2, axis=1, keepdims=True) / (E - 1)
        loss_ref[...] = LOSS_COEF * var / (mean * mean + 1e-10)


def kernel(x, mask, w_gate, W1, b1, W2, b2):
    B, S, D = x.shape
    E = w_gate.shape[1]
    H = W1.shape[2]
    Sb = 512
    T = S // Sb
    xs = x.reshape(S, D)
    maskf = mask.reshape(S, 1).astype(jnp.float32)
    b1r = b1.reshape(E, 1, H)
    b2r = b2.reshape(E, 1, D)

    y, loss = pl.pallas_call(
        functools.partial(_moe_body, E=E, T=T),
        grid=(T, E),
        in_specs=[
            pl.BlockSpec((Sb, D), lambda t, e: (t, 0)),
            pl.BlockSpec((Sb, 1), lambda t, e: (t, 0)),
            pl.BlockSpec((D, E), lambda t, e: (0, 0)),
            pl.BlockSpec((1, D, H), lambda t, e: (e, 0, 0)),
            pl.BlockSpec((1, 1, H), lambda t, e: (e, 0, 0)),
            pl.BlockSpec((1, H, D), lambda t, e: (e, 0, 0)),
            pl.BlockSpec((1, 1, D), lambda t, e: (e, 0, 0)),
        ],
        out_specs=[
            pl.BlockSpec((Sb, D), lambda t, e: (t, 0)),
            pl.BlockSpec((1, 1), lambda t, e: (0, 0)),
        ],
        out_shape=[
            jax.ShapeDtypeStruct((S, D), jnp.float32),
            jax.ShapeDtypeStruct((1, 1), jnp.float32),
        ],
        scratch_shapes=[
            pltpu.VMEM((Sb, E), jnp.float32),
            pltpu.VMEM((1, E), jnp.float32),
        ],
        compiler_params=pltpu.CompilerParams(
            dimension_semantics=("arbitrary", "arbitrary"),
        ),
    )(xs, maskf, w_gate, W1, b1r, W2, b2r)

    return y.reshape(B, S, D), loss[0, 0]
